# Initial kernel scaffold; baseline (speedup 1.0000x reference)
import jax
import jax.numpy as jnp
from jax.experimental import pallas as pl


def kernel(data, ivectors):
    raise NotImplementedError("write your pallas kernel here")


# speedup vs baseline: 1.3818x; 1.3818x over previous
import functools

import jax
import jax.numpy as jnp
from jax import lax
from jax.experimental import pallas as pl
from jax.experimental.pallas import tpu as pltpu
from jax.experimental.pallas import tpu_sc as plsc

VOCAB = 10000
D = 300
DP = 304
B = 4096 * 50
NC, NS = 2, 16
NW = NC * NS
BPW = B // NW
CHUNK = 128
NCHUNK = BPW // CHUNK


def _sc_gather(table, idx2d):
  mesh = plsc.VectorSubcoreMesh(core_axis_name="c", subcore_axis_name="s")

  @functools.partial(
      pl.kernel,
      mesh=mesh,
      out_type=jax.ShapeDtypeStruct((B, DP), jnp.float32),
      scratch_types=[
          pltpu.VMEM((NCHUNK, CHUNK), jnp.int32),
          pltpu.VMEM((2, CHUNK, DP), jnp.float32),
          pltpu.SemaphoreType.DMA,
          pltpu.SemaphoreType.DMA,
      ],
      compiler_params=pltpu.CompilerParams(use_tc_tiling_on_sc=False),
  )
  def k(table_hbm, idx_hbm, out_hbm, idx_v, rows_v, gsem, ssem):
    wid = lax.axis_index("s") * NC + lax.axis_index("c")
    base = wid * BPW
    pltpu.sync_copy(idx_hbm.at[pl.ds(wid * NCHUNK, NCHUNK)], idx_v)

    def body(g, carry):
      cbase = g * CHUNK
      pltpu.async_copy(
          table_hbm.at[idx_v.at[g]], rows_v.at[0], gsem
      ).wait()
      pltpu.async_copy(
          rows_v.at[0], out_hbm.at[pl.ds(base + cbase, CHUNK)], ssem
      ).wait()
      return carry

    lax.fori_loop(0, NCHUNK, body, 0)

  return k(table, idx2d)


def kernel(data, ivectors):
  table = jnp.pad(ivectors, ((0, 0), (0, DP - D)))
  idx2d = data.reshape(B // CHUNK, CHUNK).astype(jnp.int32)
  out = _sc_gather(table, idx2d)
  return out[:, :D].reshape(data.shape[0], data.shape[1], D)
